# Initial kernel scaffold; baseline (speedup 1.0000x reference)
import jax
import jax.numpy as jnp
from jax.experimental import pallas as pl


def kernel(x, d0_w1, d0_s1, d0_b1, d0_w2, d0_s2, d0_b2, d1_w1, d1_s1, d1_b1, d1_w2, d1_s2, d1_b2, d2_w1, d2_s1, d2_b1, d2_w2, d2_s2, d2_b2, d3_w1, d3_s1, d3_b1, d3_w2, d3_s2, d3_b2, last_w1, last_s1, last_b1, last_w2, last_s2, last_b2, u0_wt, u0_bt, u0_w1, u0_s1, u0_b1, u0_w2, u0_s2, u0_b2, u1_wt, u1_bt, u1_w1, u1_s1, u1_b1, u1_w2, u1_s2, u1_b2, u2_wt, u2_bt, u2_w1, u2_s1, u2_b1, u2_w2, u2_s2, u2_b2, u3_wt, u3_bt, u3_w1, u3_s1, u3_b1, u3_w2, u3_s2, u3_b2, final_w, final_b):
    raise NotImplementedError("write your pallas kernel here")


# speedup vs baseline: 1.1930x; 1.1930x over previous
import functools

import jax
import jax.numpy as jnp
from jax.experimental import pallas as pl
from jax.experimental.pallas import tpu as pltpu


def _fill_pad(pad_ref, x, H, W, cin):
    pad_ref[...] = jnp.zeros_like(pad_ref)
    pad_ref[1:H + 1, 1:W + 1, :cin] = x.astype(pad_ref.dtype)


def _im2col(pat_ref, pad_ref, H, W, cin):
    for t in range(9):
        ki, kj = divmod(t, 3)
        pat_ref[:, :, t * cin:(t + 1) * cin] = (
            pad_ref[ki:ki + H, kj:kj + W, :cin])


def _chunks(H):
    c = 16 if H >= 32 else H
    return [(h, c) for h in range(0, H, c)]


def _conv_dot(pat_ref, w_ref, cin, h0, hn):
    return jax.lax.dot_general(
        pat_ref[h0:h0 + hn, :, :9 * cin], w_ref[...],
        dimension_numbers=(((2,), (0,)), ((), ())),
        preferred_element_type=jnp.float32)


def _bn_relu(acc, s_ref, b_ref):
    return jnp.maximum(acc * s_ref[...] + b_ref[...], 0.0)


def _pool(y):
    H, W, C = y.shape
    y = jnp.max(y.reshape(H, W // 2, 2, C), axis=2)
    return jnp.max(y.reshape(H // 2, 2, W // 2, C), axis=1)


def _enc_stage(pad1, pad2, pat, out_ref, w1, s1, b1, w2, s2, b2,
               H, W, cin, cmid):
    _im2col(pat, pad1, H, W, cin)
    pad2[...] = jnp.zeros_like(pad2)
    for h0, hn in _chunks(H):
        y1 = _bn_relu(_conv_dot(pat, w1, cin, h0, hn), s1, b1)
        pad2[1 + h0:1 + h0 + hn, 1:W + 1, :] = y1
    _im2col(pat, pad2, H, W, cmid)
    for h0, hn in _chunks(H):
        y2 = _bn_relu(_conv_dot(pat, w2, cmid, h0, hn), s2, b2)
        pad2[1 + h0:1 + h0 + hn, 1:W + 1, :] = y2
        out_ref[0, h0:h0 + hn] = y2


def _enc_kernel(S, feats, cin0,
                x_ref,
                e0w1, e0s1, e0b1, e0w2, e0s2, e0b2,
                e1w1, e1s1, e1b1, e1w2, e1s2, e1b2,
                e2w1, e2s1, e2b1, e2w2, e2s2, e2b2,
                e3w1, e3s1, e3b1, e3w2, e3s2, e3b2,
                e4w1, e4s1, e4b1, e4w2, e4s2, e4b2,
                s0_ref, s1_ref, s2_ref, s3_ref, b_ref,
                p1_0, p2_0, p1_1, p2_1, p1_2, p2_2, p1_3, p2_3, p1_4, p2_4,
                q0, q1, q2, q3, q4):
    f0, f1, f2, f3 = feats
    n = pl.program_id(0)
    p1_0[...] = jnp.zeros_like(p1_0)
    for c in range(cin0):
        p1_0[1:S + 1, 1:S + 1, c] = x_ref[n, c * S:(c + 1) * S, :]
    _enc_stage(p1_0, p2_0, q0, s0_ref, e0w1, e0s1, e0b1, e0w2, e0s2, e0b2,
               S, S, cin0, f0)
    _fill_pad(p1_1, _pool(p2_0[1:S + 1, 1:S + 1, :]), S // 2, S // 2, f0)
    _enc_stage(p1_1, p2_1, q1, s1_ref, e1w1, e1s1, e1b1, e1w2, e1s2, e1b2,
               S // 2, S // 2, f0, f1)
    _fill_pad(p1_2, _pool(p2_1[1:S // 2 + 1, 1:S // 2 + 1, :]),
              S // 4, S // 4, f1)
    _enc_stage(p1_2, p2_2, q2, s2_ref, e2w1, e2s1, e2b1, e2w2, e2s2, e2b2,
               S // 4, S // 4, f1, f2)
    _fill_pad(p1_3, _pool(p2_2[1:S // 4 + 1, 1:S // 4 + 1, :]),
              S // 8, S // 8, f2)
    _enc_stage(p1_3, p2_3, q3, s3_ref, e3w1, e3s1, e3b1, e3w2, e3s2, e3b2,
               S // 8, S // 8, f2, f3)
    _fill_pad(p1_4, _pool(p2_3[1:S // 8 + 1, 1:S // 8 + 1, :]),
              S // 16, S // 16, f3)
    _enc_stage(p1_4, p2_4, q4, b_ref, e4w1, e4s1, e4b1, e4w2, e4s2, e4b2,
               S // 16, S // 16, f3, 2 * f3)


def _dec_stage(h, s_ref, skip_pad, up_pad, pat,
               wt, bt, w1s, w1u, s1, b1, w2, s2, b2, Hh, Wh, F):
    H, W = 2 * Hh, 2 * Wh
    _fill_pad(skip_pad, s_ref[0], H, W, F)
    taps = jax.lax.dot_general(
        h, wt[...], dimension_numbers=(((2,), (0,)), ((), ())),
        preferred_element_type=jnp.float32) + bt[...]
    up_pad[...] = jnp.zeros_like(up_pad)
    for a in (0, 1):
        ta = taps[:, :, (2 * a) * F:(2 * a + 1) * F]
        tb = taps[:, :, (2 * a + 1) * F:(2 * a + 2) * F]
        for i in range(Hh):
            row = jnp.concatenate(
                [ta[i][:, None, :], tb[i][:, None, :]], axis=1)
            up_pad[1 + a + 2 * i, 1:W + 1, :] = (
                row.reshape(W, F).astype(up_pad.dtype))
    _im2col(pat, skip_pad, H, W, F)
    for h0, hn in _chunks(H):
        skip_pad[1 + h0:1 + h0 + hn, 1:W + 1, :] = (
            _conv_dot(pat, w1s, F, h0, hn))
    _im2col(pat, up_pad, H, W, F)
    for h0, hn in _chunks(H):
        acc = (skip_pad[1 + h0:1 + h0 + hn, 1:W + 1, :]
               + _conv_dot(pat, w1u, F, h0, hn))
        skip_pad[1 + h0:1 + h0 + hn, 1:W + 1, :] = _bn_relu(acc, s1, b1)
    _im2col(pat, skip_pad, H, W, F)
    outs = []
    for h0, hn in _chunks(H):
        outs.append(_bn_relu(_conv_dot(pat, w2, F, h0, hn), s2, b2))
    return outs[0] if len(outs) == 1 else jnp.concatenate(outs, axis=0)


def _dec_kernel(S, feats,
                s0_ref, s1_ref, s2_ref, s3_ref, b_ref,
                u0wt, u0bt, u0w1s, u0w1u, u0s1, u0b1, u0w2, u0s2, u0b2,
                u1wt, u1bt, u1w1s, u1w1u, u1s1, u1b1, u1w2, u1s2, u1b2,
                u2wt, u2bt, u2w1s, u2w1u, u2s1, u2b1, u2w2, u2s2, u2b2,
                u3wt, u3bt, u3w1s, u3w1u, u3s1, u3b1, u3w2, u3s2, u3b2,
                fw, fb,
                o_ref,
                k3, v3, k2, v2, k1, v1, k0, v0,
                q0, q1, q2, q3):
    f0, f1, f2, f3 = feats
    h = b_ref[0]
    h = _dec_stage(h, s3_ref, k3, v3, q3, u0wt, u0bt, u0w1s, u0w1u,
                   u0s1, u0b1, u0w2, u0s2, u0b2, S // 16, S // 16, f3)
    h = _dec_stage(h, s2_ref, k2, v2, q2, u1wt, u1bt, u1w1s, u1w1u,
                   u1s1, u1b1, u1w2, u1s2, u1b2, S // 8, S // 8, f2)
    h = _dec_stage(h, s1_ref, k1, v1, q1, u2wt, u2bt, u2w1s, u2w1u,
                   u2s1, u2b1, u2w2, u2s2, u2b2, S // 4, S // 4, f1)
    h = _dec_stage(h, s0_ref, k0, v0, q0, u3wt, u3bt, u3w1s, u3w1u,
                   u3s1, u3b1, u3w2, u3s2, u3b2, S // 2, S // 2, f0)
    x2 = h.reshape(S * S, f0)
    logits = jax.lax.dot_general(
        fw[...], x2, dimension_numbers=(((1,), (1,)), ((), ())),
        preferred_element_type=jnp.float32)
    logits = logits + fb[...]
    m = jnp.max(logits, axis=0, keepdims=True)
    e = jnp.exp(logits - m)
    denom = jnp.sum(e, axis=0, keepdims=True)
    o_ref[0] = (e * pl.reciprocal(denom, approx=True)).astype(o_ref.dtype)


def kernel(x, d0_w1, d0_s1, d0_b1, d0_w2, d0_s2, d0_b2,
           d1_w1, d1_s1, d1_b1, d1_w2, d1_s2, d1_b2,
           d2_w1, d2_s1, d2_b1, d2_w2, d2_s2, d2_b2,
           d3_w1, d3_s1, d3_b1, d3_w2, d3_s2, d3_b2,
           last_w1, last_s1, last_b1, last_w2, last_s2, last_b2,
           u0_wt, u0_bt, u0_w1, u0_s1, u0_b1, u0_w2, u0_s2, u0_b2,
           u1_wt, u1_bt, u1_w1, u1_s1, u1_b1, u1_w2, u1_s2, u1_b2,
           u2_wt, u2_bt, u2_w1, u2_s1, u2_b1, u2_w2, u2_s2, u2_b2,
           u3_wt, u3_bt, u3_w1, u3_s1, u3_b1, u3_w2, u3_s2, u3_b2,
           final_w, final_b):
    N, cin0, S, _ = x.shape
    feats = (d0_w1.shape[-1], d1_w1.shape[-1], d2_w1.shape[-1],
             d3_w1.shape[-1])
    f0, f1, f2, f3 = feats
    n_cls = final_w.shape[-1]
    wspec = pl.BlockSpec(memory_space=pltpu.MemorySpace.VMEM)

    def cw(w):
        return w.reshape(9 * w.shape[2], w.shape[3])

    def sb(v):
        return v.reshape(1, -1)

    enc_ops = [x.reshape(N, cin0 * S, S)]
    for p in ((d0_w1, d0_s1, d0_b1, d0_w2, d0_s2, d0_b2),
              (d1_w1, d1_s1, d1_b1, d1_w2, d1_s2, d1_b2),
              (d2_w1, d2_s1, d2_b1, d2_w2, d2_s2, d2_b2),
              (d3_w1, d3_s1, d3_b1, d3_w2, d3_s2, d3_b2),
              (last_w1, last_s1, last_b1, last_w2, last_s2, last_b2)):
        w1, s1, b1, w2, s2, b2 = p
        enc_ops += [cw(w1), sb(s1), sb(b1), cw(w2), sb(s2), sb(b2)]

    enc_scr = []
    cs1 = [cin0, f0, f1, f2, f3]
    cs2 = [f0, f1, f2, f3, 2 * f3]
    ss = [S, S // 2, S // 4, S // 8, S // 16]
    for s, c1, c2 in zip(ss, cs1, cs2):
        enc_scr.append(pltpu.VMEM((s + 2, s + 2, c1), jnp.float32))
        enc_scr.append(pltpu.VMEM((s + 2, s + 2, c2), jnp.float32))
    enc_scr = [enc_scr[0], enc_scr[1], enc_scr[2], enc_scr[3],
               enc_scr[4], enc_scr[5], enc_scr[6], enc_scr[7],
               enc_scr[8], enc_scr[9]]
    for s, c in zip(ss, cs2):
        enc_scr.append(pltpu.VMEM((s, s, 9 * c), jnp.float32))

    skip_shapes = [jax.ShapeDtypeStruct((N, s, s, c), jnp.float32)
                   for s, c in zip(ss, cs2)]
    skips = pl.pallas_call(
        functools.partial(_enc_kernel, S, feats, cin0),
        out_shape=tuple(skip_shapes),
        grid=(N,),
        in_specs=[wspec for _ in enc_ops],
        out_specs=tuple(
            pl.BlockSpec((1, s, s, c), lambda n: (n, 0, 0, 0))
            for s, c in zip(ss, cs2)),
        scratch_shapes=enc_scr,
        compiler_params=pltpu.CompilerParams(
            dimension_semantics=("parallel",)),
    )(*enc_ops)
    s0, s1_, s2_, s3_, b = skips

    dec_ops = [s0, s1_, s2_, s3_, b]
    for (wt, bt, w1, s1, b1, w2, s2, b2) in (
            (u0_wt, u0_bt, u0_w1, u0_s1, u0_b1, u0_w2, u0_s2, u0_b2),
            (u1_wt, u1_bt, u1_w1, u1_s1, u1_b1, u1_w2, u1_s2, u1_b2),
            (u2_wt, u2_bt, u2_w1, u2_s1, u2_b1, u2_w2, u2_s2, u2_b2),
            (u3_wt, u3_bt, u3_w1, u3_s1, u3_b1, u3_w2, u3_s2, u3_b2)):
        F = wt.shape[-1]
        wt2d = jnp.transpose(wt, (2, 0, 1, 3)).reshape(2 * F, 4 * F)
        bt4 = jnp.tile(bt, 4).reshape(1, 4 * F)
        w1s = cw(w1[:, :, :F, :])
        w1u = cw(w1[:, :, F:, :])
        dec_ops += [wt2d, bt4, w1s, w1u, sb(s1), sb(b1),
                    cw(w2), sb(s2), sb(b2)]
    dec_ops += [jnp.transpose(final_w, (1, 0)), final_b.reshape(n_cls, 1)]

    dec_in_specs = [
        pl.BlockSpec((1, s, s, c), lambda n: (n, 0, 0, 0))
        for s, c in zip(ss, cs2)]
    dec_in_specs += [wspec for _ in dec_ops[5:]]

    dec_scr = []
    for s, c in ((S // 16, f3), (S // 8, f2), (S // 4, f1), (S // 2, f0)):
        dec_scr.append(pltpu.VMEM((2 * s + 2, 2 * s + 2, c), jnp.float32))
        dec_scr.append(pltpu.VMEM((2 * s + 2, 2 * s + 2, c), jnp.float32))
    for s, c in ((S, f0), (S // 2, f1), (S // 4, f2), (S // 8, f3)):
        dec_scr.append(pltpu.VMEM((s, s, 9 * c), jnp.float32))

    y = pl.pallas_call(
        functools.partial(_dec_kernel, S, feats),
        out_shape=jax.ShapeDtypeStruct((N, n_cls, S * S), jnp.float32),
        grid=(N,),
        in_specs=dec_in_specs,
        out_specs=pl.BlockSpec((1, n_cls, S * S), lambda n: (n, 0, 0)),
        scratch_shapes=dec_scr,
        compiler_params=pltpu.CompilerParams(
            dimension_semantics=("parallel",)),
    )(*dec_ops)
    return y.reshape(N, n_cls, S, S)
